# Initial kernel scaffold; baseline (speedup 1.0000x reference)
import jax
import jax.numpy as jnp
from jax.experimental import pallas as pl


def kernel(nodes, neigh_idx, features, weight):
    raise NotImplementedError("write your pallas kernel here")


# speedup vs baseline: 1.9847x; 1.9847x over previous
import functools

import jax
import jax.numpy as jnp
from jax import lax
from jax.experimental import pallas as pl
from jax.experimental.pallas import tpu as pltpu
from jax.experimental.pallas import tpu_sc as plsc

_NW = 32
_CHUNK = 8


def _gather_combine(nodes, neigh_flat, features, B, S, D):
    n_chunks = B // _CHUNK
    mesh = plsc.VectorSubcoreMesh(
        core_axis_name="c", subcore_axis_name="s", num_cores=2, num_subcores=16
    )
    cs = _CHUNK * S

    @functools.partial(
        pl.kernel,
        out_type=jax.ShapeDtypeStruct((B, 2 * D), jnp.float32),
        mesh=mesh,
        scratch_types=[
            pltpu.VMEM((cs + _CHUNK,), jnp.int32),
            pltpu.VMEM((cs + _CHUNK, D), jnp.float32),
            pltpu.VMEM((_CHUNK, 2 * D), jnp.float32),
            pltpu.SemaphoreType.DMA,
        ],
    )
    def k(nodes_hbm, neigh_hbm, feat_hbm, out_hbm, idx_v, rows_v, out_v, sem):
        wid = lax.axis_index("s") * 2 + lax.axis_index("c")
        n_my = (n_chunks - wid + _NW - 1) // _NW

        def body(j, carry):
            base = (wid + j * _NW) * _CHUNK
            pltpu.sync_copy(neigh_hbm.at[pl.ds(base * S, cs)], idx_v.at[pl.ds(0, cs)])
            pltpu.sync_copy(nodes_hbm.at[pl.ds(base, _CHUNK)], idx_v.at[pl.ds(cs, _CHUNK)])
            pltpu.async_copy(feat_hbm.at[idx_v], rows_v, sem).wait()
            for c in range(_CHUNK):
                for dd in range(D // 16):
                    sl = pl.ds(dd * 16, 16)
                    out_v[c, sl] = rows_v[cs + c, sl]
                    acc = rows_v[c * S, sl]
                    for s in range(1, S):
                        acc = acc + rows_v[c * S + s, sl]
                    out_v[c, pl.ds(D + dd * 16, 16)] = acc * (1.0 / S)
            pltpu.sync_copy(out_v, out_hbm.at[pl.ds(base, _CHUNK)])
            return carry

        lax.fori_loop(0, n_my, body, 0)

    return k(nodes, neigh_flat, features)


def _matmul_relu(combined, weight, B, D, E):
    R = 2000

    def mm(x_ref, w_ref, o_ref):
        acc = lax.dot_general(
            x_ref[...], w_ref[...], (((1,), (1,)), ((), ())),
            preferred_element_type=jnp.float32,
        )
        o_ref[...] = jnp.maximum(acc, 0.0)

    return pl.pallas_call(
        mm,
        grid=(B // R,),
        in_specs=[
            pl.BlockSpec((R, 2 * D), lambda i: (i, 0)),
            pl.BlockSpec((E, 2 * D), lambda i: (0, 0)),
        ],
        out_specs=pl.BlockSpec((R, E), lambda i: (i, 0)),
        out_shape=jax.ShapeDtypeStruct((B, E), jnp.float32),
    )(combined, weight)


def kernel(nodes, neigh_idx, features, weight):
    B, S = neigh_idx.shape
    D = features.shape[1]
    E = weight.shape[0]
    nodes = nodes.astype(jnp.int32)
    neigh_flat = neigh_idx.astype(jnp.int32).reshape(-1)
    combined = _gather_combine(nodes, neigh_flat, features, B, S, D)
    return _matmul_relu(combined, weight, B, D, E)


# speedup vs baseline: 2.9308x; 1.4767x over previous
import functools

import jax
import jax.numpy as jnp
from jax import lax
from jax.experimental import pallas as pl
from jax.experimental.pallas import tpu as pltpu
from jax.experimental.pallas import tpu_sc as plsc

_NW = 32
_C = 8
_NCH = 80


def _gather_combine(nodes2d, neigh2d, features, B, S, D):
    n_real = B // _C
    cs = _C * S
    mesh = plsc.VectorSubcoreMesh(
        core_axis_name="c", subcore_axis_name="s", num_cores=2, num_subcores=16
    )

    @functools.partial(
        pl.kernel,
        out_type=jax.ShapeDtypeStruct((B, 2 * D), jnp.float32),
        mesh=mesh,
        scratch_types=[
            pltpu.VMEM((_NCH, _C), jnp.int32),
            pltpu.VMEM((_NCH, cs), jnp.int32),
            pltpu.VMEM((cs, D), jnp.float32),
            pltpu.VMEM((cs, D), jnp.float32),
            pltpu.VMEM((_C, D), jnp.float32),
            pltpu.VMEM((_C, D), jnp.float32),
            pltpu.VMEM((_C, 2 * D), jnp.float32),
            pltpu.VMEM((_C, 2 * D), jnp.float32),
            pltpu.SemaphoreType.DMA,
            pltpu.SemaphoreType.DMA,
            pltpu.SemaphoreType.DMA,
            pltpu.SemaphoreType.DMA,
            pltpu.SemaphoreType.DMA,
            pltpu.SemaphoreType.DMA,
        ],
    )
    def k(nodes_hbm, neigh_hbm, feat_hbm, out_hbm, idxs_v, idxn_v,
          rn0, rn1, rs0, rs1, ov0, ov1,
          semn0, semn1, sems0, sems1, semo0, semo1):
        rn = (rn0, rn1)
        rs = (rs0, rs1)
        ov = (ov0, ov1)
        semn = (semn0, semn1)
        sems = (sems0, sems1)
        semo = (semo0, semo1)

        wid = lax.axis_index("s") * 2 + lax.axis_index("c")
        n_my = jnp.minimum(_NCH, jnp.maximum(0, n_real - _NCH * wid))
        row0 = wid * (_NCH * _C)

        pltpu.sync_copy(nodes_hbm.at[pl.ds(wid * _NCH, _NCH)], idxs_v)
        pltpu.sync_copy(neigh_hbm.at[pl.ds(wid * _NCH, _NCH)], idxn_v)

        def fire(c, b):
            cc = jnp.minimum(c, _NCH - 1)

            @pl.when(c < n_my)
            def _():
                pltpu.async_copy(feat_hbm.at[idxn_v.at[cc]], rn[b], semn[b])
                pltpu.async_copy(feat_hbm.at[idxs_v.at[cc]], rs[b], sems[b])

        def wait_gather(c, b):
            cc = jnp.minimum(c, _NCH - 1)

            @pl.when(c < n_my)
            def _():
                pltpu.make_async_copy(feat_hbm.at[idxn_v.at[cc]], rn[b], semn[b]).wait()
                pltpu.make_async_copy(feat_hbm.at[idxs_v.at[cc]], rs[b], sems[b]).wait()

        def wait_store(c, b):
            @pl.when((c >= 0) & (c < n_my))
            def _():
                pltpu.make_async_copy(ov[b], out_hbm.at[pl.ds(0, _C)], semo[b]).wait()

        def compute_store(c, b):
            @pl.when(c < n_my)
            def _():
                for r in range(_C):
                    for dd in range(D // 16):
                        sl = pl.ds(dd * 16, 16)
                        ov[b][r, sl] = rs[b][r, sl]
                        acc = rn[b][r * S, sl]
                        for s in range(1, S):
                            acc = acc + rn[b][r * S + s, sl]
                        ov[b][r, pl.ds(D + dd * 16, 16)] = acc * (1.0 / S)
                pltpu.async_copy(ov[b], out_hbm.at[pl.ds(row0 + c * _C, _C)], semo[b])

        fire(jnp.int32(0), 0)
        fire(jnp.int32(1), 1)

        def body(i, carry):
            jo = i * 2
            for b in range(2):
                c = jo + b
                wait_gather(c, b)
                wait_store(c - 2, b)
                compute_store(c, b)
                fire(c + 2, b)
            return carry

        lax.fori_loop(0, _NCH // 2, body, 0)
        wait_store(jnp.int32(_NCH - 2), 0)
        wait_store(jnp.int32(_NCH - 1), 1)

    return k(nodes2d, neigh2d, features)


def _matmul_relu(combined, weight, B, D, E):
    R = 2000

    def mm(x_ref, w_ref, o_ref):
        acc = lax.dot_general(
            x_ref[...], w_ref[...], (((1,), (1,)), ((), ())),
            preferred_element_type=jnp.float32,
        )
        o_ref[...] = jnp.maximum(acc, 0.0)

    return pl.pallas_call(
        mm,
        grid=(B // R,),
        in_specs=[
            pl.BlockSpec((R, 2 * D), lambda i: (i, 0)),
            pl.BlockSpec((E, 2 * D), lambda i: (0, 0)),
        ],
        out_specs=pl.BlockSpec((R, E), lambda i: (i, 0)),
        out_shape=jax.ShapeDtypeStruct((B, E), jnp.float32),
    )(combined, weight)


def kernel(nodes, neigh_idx, features, weight):
    B, S = neigh_idx.shape
    D = features.shape[1]
    E = weight.shape[0]
    b_pad = _NW * _NCH * _C
    nodes_p = jnp.pad(nodes.astype(jnp.int32), (0, b_pad - B)).reshape(-1, _C)
    neigh_p = jnp.pad(
        neigh_idx.astype(jnp.int32).reshape(-1), (0, (b_pad - B) * S)
    ).reshape(-1, _C * S)
    combined = _gather_combine(nodes_p, neigh_p, features, B, S, D)
    return _matmul_relu(combined, weight, B, D, E)


# speedup vs baseline: 4.6724x; 1.5942x over previous
import functools

import jax
import jax.numpy as jnp
from jax import lax
from jax.experimental import pallas as pl
from jax.experimental.pallas import tpu as pltpu
from jax.experimental.pallas import tpu_sc as plsc

_NW = 32
_C = 8
_NCH = 80
_NBUF = 4


def _gather_combine(nodes2d, neigh2d, features, B, S, D):
    n_real = B // _C
    cs = _C * S
    mesh = plsc.VectorSubcoreMesh(
        core_axis_name="c", subcore_axis_name="s", num_cores=2, num_subcores=16
    )

    @functools.partial(
        pl.kernel,
        out_type=jax.ShapeDtypeStruct((B, 2 * D), jnp.float32),
        mesh=mesh,
        scratch_types=[
            pltpu.VMEM((_NCH, _C), jnp.int32),
            pltpu.VMEM((_NCH, cs), jnp.int32),
        ]
        + [pltpu.VMEM((cs, D), jnp.float32)] * _NBUF
        + [pltpu.VMEM((_C, D), jnp.float32)] * _NBUF
        + [pltpu.VMEM((_C, 2 * D), jnp.float32)] * _NBUF
        + [pltpu.SemaphoreType.DMA] * (3 * _NBUF),
    )
    def k(nodes_hbm, neigh_hbm, feat_hbm, out_hbm, idxs_v, idxn_v, *bufs):
        rn = bufs[0:_NBUF]
        rs = bufs[_NBUF:2 * _NBUF]
        ov = bufs[2 * _NBUF:3 * _NBUF]
        semn = bufs[3 * _NBUF:4 * _NBUF]
        sems = bufs[4 * _NBUF:5 * _NBUF]
        semo = bufs[5 * _NBUF:6 * _NBUF]

        wid = lax.axis_index("s") * 2 + lax.axis_index("c")
        n_my = jnp.minimum(_NCH, jnp.maximum(0, n_real - _NCH * wid))
        row0 = wid * (_NCH * _C)

        pltpu.sync_copy(nodes_hbm.at[pl.ds(wid * _NCH, _NCH)], idxs_v)
        pltpu.sync_copy(neigh_hbm.at[pl.ds(wid * _NCH, _NCH)], idxn_v)

        def fire(c, b):
            cc = jnp.minimum(c, _NCH - 1)

            @pl.when(c < n_my)
            def _():
                pltpu.async_copy(feat_hbm.at[idxn_v.at[cc]], rn[b], semn[b])
                pltpu.async_copy(feat_hbm.at[idxs_v.at[cc]], rs[b], sems[b])

        def wait_gather(c, b):
            cc = jnp.minimum(c, _NCH - 1)

            @pl.when(c < n_my)
            def _():
                pltpu.make_async_copy(feat_hbm.at[idxn_v.at[cc]], rn[b], semn[b]).wait()
                pltpu.make_async_copy(feat_hbm.at[idxs_v.at[cc]], rs[b], sems[b]).wait()

        def wait_store(c, b):
            @pl.when((c >= 0) & (c < n_my))
            def _():
                pltpu.make_async_copy(ov[b], out_hbm.at[pl.ds(0, _C)], semo[b]).wait()

        def compute_store(c, b):
            @pl.when(c < n_my)
            def _():
                def row(r, carry):
                    for dd in range(D // 16):
                        sl = pl.ds(dd * 16, 16)
                        ov[b][r, sl] = rs[b][r, sl]
                        acc = rn[b][r * S, sl]
                        for s in range(1, S):
                            acc = acc + rn[b][r * S + s, sl]
                        ov[b][r, pl.ds(D + dd * 16, 16)] = acc * (1.0 / S)
                    return carry

                lax.fori_loop(0, _C, row, 0)
                pltpu.async_copy(ov[b], out_hbm.at[pl.ds(row0 + c * _C, _C)], semo[b])

        for b in range(_NBUF):
            fire(jnp.int32(b), b)

        def body(i, carry):
            jo = i * _NBUF
            for b in range(_NBUF):
                c = jo + b
                wait_gather(c, b)
                wait_store(c - _NBUF, b)
                compute_store(c, b)
                fire(c + _NBUF, b)
            return carry

        lax.fori_loop(0, _NCH // _NBUF, body, 0)
        for b in range(_NBUF):
            wait_store(jnp.int32(_NCH - _NBUF + b), b)

    return k(nodes2d, neigh2d, features)


def _matmul_relu(combined, weight, B, D, E):
    R = 2000

    def mm(x_ref, w_ref, o_ref):
        acc = lax.dot_general(
            x_ref[...], w_ref[...], (((1,), (1,)), ((), ())),
            preferred_element_type=jnp.float32,
        )
        o_ref[...] = jnp.maximum(acc, 0.0)

    return pl.pallas_call(
        mm,
        grid=(B // R,),
        in_specs=[
            pl.BlockSpec((R, 2 * D), lambda i: (i, 0)),
            pl.BlockSpec((E, 2 * D), lambda i: (0, 0)),
        ],
        out_specs=pl.BlockSpec((R, E), lambda i: (i, 0)),
        out_shape=jax.ShapeDtypeStruct((B, E), jnp.float32),
    )(combined, weight)


def kernel(nodes, neigh_idx, features, weight):
    B, S = neigh_idx.shape
    D = features.shape[1]
    E = weight.shape[0]
    b_pad = _NW * _NCH * _C
    nodes_p = jnp.pad(nodes.astype(jnp.int32), (0, b_pad - B)).reshape(-1, _C)
    neigh_p = jnp.pad(
        neigh_idx.astype(jnp.int32).reshape(-1), (0, (b_pad - B) * S)
    ).reshape(-1, _C * S)
    combined = _gather_combine(nodes_p, neigh_p, features, B, S, D)
    return _matmul_relu(combined, weight, B, D, E)


# speedup vs baseline: 5.0698x; 1.0851x over previous
import functools

import jax
import jax.numpy as jnp
from jax import lax
from jax.experimental import pallas as pl
from jax.experimental.pallas import tpu as pltpu
from jax.experimental.pallas import tpu_sc as plsc

_NW = 32
_C = 8
_NCH = 80
_NBUF = 4


def _gather_combine(nodes2d, neigh2d, features, B, S, D):
    n_real = B // _C
    cs = _C * S
    mesh = plsc.VectorSubcoreMesh(
        core_axis_name="c", subcore_axis_name="s", num_cores=2, num_subcores=16
    )

    @functools.partial(
        pl.kernel,
        out_type=jax.ShapeDtypeStruct((B, 2 * D), jnp.float32),
        mesh=mesh,
        scratch_types=[
            pltpu.VMEM((_NCH, _C), jnp.int32),
            pltpu.VMEM((_NCH, cs), jnp.int32),
        ]
        + [pltpu.VMEM((cs, D), jnp.float32)] * _NBUF
        + [pltpu.VMEM((_C, D), jnp.float32)] * _NBUF
        + [pltpu.VMEM((_C, 2 * D), jnp.float32)] * _NBUF
        + [pltpu.SemaphoreType.DMA] * (3 * _NBUF),
    )
    def k(nodes_hbm, neigh_hbm, feat_hbm, out_hbm, idxs_v, idxn_v, *bufs):
        rn = bufs[0:_NBUF]
        rs = bufs[_NBUF:2 * _NBUF]
        ov = bufs[2 * _NBUF:3 * _NBUF]
        semn = bufs[3 * _NBUF:4 * _NBUF]
        sems = bufs[4 * _NBUF:5 * _NBUF]
        semo = bufs[5 * _NBUF:6 * _NBUF]

        wid = lax.axis_index("s") * 2 + lax.axis_index("c")
        n_my = jnp.minimum(_NCH, jnp.maximum(0, n_real - _NCH * wid))
        row0 = wid * (_NCH * _C)

        pltpu.sync_copy(nodes_hbm.at[pl.ds(wid * _NCH, _NCH)], idxs_v)
        pltpu.sync_copy(neigh_hbm.at[pl.ds(wid * _NCH, _NCH)], idxn_v)

        def fire(c, b):
            cc = jnp.minimum(c, _NCH - 1)

            @pl.when(c < n_my)
            def _():
                pltpu.async_copy(feat_hbm.at[idxn_v.at[cc]], rn[b], semn[b])
                pltpu.async_copy(feat_hbm.at[idxs_v.at[cc]], rs[b], sems[b])

        def wait_gather(c, b):
            cc = jnp.minimum(c, _NCH - 1)

            @pl.when(c < n_my)
            def _():
                pltpu.make_async_copy(feat_hbm.at[idxn_v.at[cc]], rn[b], semn[b]).wait()
                pltpu.make_async_copy(feat_hbm.at[idxs_v.at[cc]], rs[b], sems[b]).wait()

        def wait_store(c, b):
            @pl.when((c >= 0) & (c < n_my))
            def _():
                pltpu.make_async_copy(ov[b], out_hbm.at[pl.ds(0, _C)], semo[b]).wait()

        def compute_store(c, b):
            @pl.when(c < n_my)
            def _():
                def row(rp, carry):
                    for rr in range(2):
                        r = rp * 2 + rr
                        for dd in range(D // 16):
                            sl = pl.ds(dd * 16, 16)
                            ov[b][r, sl] = rs[b][r, sl]
                            t = [
                                rn[b][r * S + 2 * s, sl] + rn[b][r * S + 2 * s + 1, sl]
                                for s in range(S // 2)
                            ]
                            while len(t) > 1:
                                t = [
                                    t[2 * i] + t[2 * i + 1] for i in range(len(t) // 2)
                                ]
                            ov[b][r, pl.ds(D + dd * 16, 16)] = t[0]
                    return carry

                lax.fori_loop(0, _C // 2, row, 0)
                pltpu.async_copy(ov[b], out_hbm.at[pl.ds(row0 + c * _C, _C)], semo[b])

        for b in range(_NBUF):
            fire(jnp.int32(b), b)

        def body(i, carry):
            jo = i * _NBUF
            for b in range(_NBUF):
                c = jo + b
                wait_gather(c, b)
                wait_store(c - _NBUF, b)
                compute_store(c, b)
                fire(c + _NBUF, b)
            return carry

        lax.fori_loop(0, _NCH // _NBUF, body, 0)
        for b in range(_NBUF):
            wait_store(jnp.int32(_NCH - _NBUF + b), b)

    return k(nodes2d, neigh2d, features)


def _matmul_relu(combined, weight, B, D, E):
    R = 2000

    def mm(x_ref, w_ref, o_ref):
        acc = lax.dot_general(
            x_ref[...], w_ref[...], (((1,), (1,)), ((), ())),
            preferred_element_type=jnp.float32,
        )
        o_ref[...] = jnp.maximum(acc, 0.0)

    return pl.pallas_call(
        mm,
        grid=(B // R,),
        in_specs=[
            pl.BlockSpec((R, 2 * D), lambda i: (i, 0)),
            pl.BlockSpec((E, 2 * D), lambda i: (0, 0)),
        ],
        out_specs=pl.BlockSpec((R, E), lambda i: (i, 0)),
        out_shape=jax.ShapeDtypeStruct((B, E), jnp.float32),
    )(combined, weight)


def kernel(nodes, neigh_idx, features, weight):
    B, S = neigh_idx.shape
    D = features.shape[1]
    E = weight.shape[0]
    b_pad = _NW * _NCH * _C
    nodes_p = jnp.pad(nodes.astype(jnp.int32), (0, b_pad - B)).reshape(-1, _C)
    neigh_p = jnp.pad(
        neigh_idx.astype(jnp.int32).reshape(-1), (0, (b_pad - B) * S)
    ).reshape(-1, _C * S)
    combined = _gather_combine(nodes_p, neigh_p, features, B, S, D)
    scale = jnp.concatenate(
        [jnp.ones((D,), jnp.float32), jnp.full((D,), 1.0 / S, jnp.float32)]
    )
    return _matmul_relu(combined, weight * scale[None, :], B, D, E)
